# Initial kernel scaffold; baseline (speedup 1.0000x reference)
import jax
import jax.numpy as jnp
from jax.experimental import pallas as pl


def kernel(gm_val, gm_index, edge_index, edge_attr, batch, W):
    raise NotImplementedError("write your pallas kernel here")


# speedup vs baseline: 2.5534x; 2.5534x over previous
import functools

import jax
import jax.numpy as jnp
from jax import lax
from jax.experimental import pallas as pl

N_GRAPHS = 256
N_PER = 64
EMB = 16
OUT = 64
TOTAL_PAIRS = N_GRAPHS * N_PER * N_PER

ROW_BLK = 8192


def _mm_body(x_ref, w_ref, o_ref):
    o_ref[...] = lax.dot_general(
        x_ref[...], w_ref[...],
        dimension_numbers=(((1,), (1,)), ((), ())),
        preferred_element_type=jnp.float32)


def _full_out_idx():
    off = jnp.repeat(jnp.arange(N_GRAPHS, dtype=jnp.int32) * N_PER, N_PER * N_PER)
    ii = jnp.tile(jnp.repeat(jnp.arange(N_PER, dtype=jnp.int32), N_PER), N_GRAPHS)
    jj = jnp.tile(jnp.arange(N_PER, dtype=jnp.int32), N_GRAPHS * N_PER)
    return jnp.stack([off + ii, off + jj])


def kernel(gm_val, gm_index, edge_index, edge_attr, batch, W):
    grid = TOTAL_PAIRS // ROW_BLK
    gm_proj = pl.pallas_call(
        _mm_body,
        grid=(grid,),
        in_specs=[
            pl.BlockSpec((ROW_BLK, EMB), lambda i: (i, 0)),
            pl.BlockSpec((OUT, EMB), lambda i: (0, 0)),
        ],
        out_specs=pl.BlockSpec((ROW_BLK, OUT), lambda i: (i, 0)),
        out_shape=jax.ShapeDtypeStruct((TOTAL_PAIRS, OUT), jnp.float32),
    )(gm_val, W)
    pos = edge_index[0] * N_PER + jnp.remainder(edge_index[1], N_PER)
    out_val = gm_proj.at[pos].add(edge_attr)
    return _full_out_idx(), out_val
